# baseline (speedup 1.0000x reference)
import dataclasses
import functools

import jax
import jax.numpy as jnp
from jax import lax
from jax.experimental import pallas as pl
from jax.experimental.pallas import tpu as pltpu
from jax.experimental.pallas import tpu_sc as plsc

_N = 100000
_E = 3200000
_DF = 128
_DE = 16
_NG = 16
_DG = 32
_NH = 128
_NO = 128

_L = 16
_NSUB = 16
_NCORE = 2
_NTILE = _NSUB * _NCORE
_NROUND = 2
_RV = _N // _NROUND
_VPR = 1563
_SCRV = _NSUB * _VPR
_W = 4000
_NW = _E // _W
_NVREG = _W // _L
_S = 1024
_U = 10
_SBLK = 400
_CNTP = 1568
_BIG = 3.0e38


def _sc_agg(src, e_attr):
    mesh = plsc.VectorSubcoreMesh(core_axis_name="c", subcore_axis_name="s")
    cp = pltpu.CompilerParams()
    if "needs_layout_passes" in pltpu.CompilerParams.__dataclass_fields__:
        cp = dataclasses.replace(cp, needs_layout_passes=False)
    cp = dataclasses.replace(cp, use_tc_tiling_on_sc=False)

    @functools.partial(
        pl.kernel,
        compiler_params=cp,
        out_type=[
            jax.ShapeDtypeStruct((_NROUND * _NTILE, _VPR * _DE),
                                 jnp.float32),
            jax.ShapeDtypeStruct((_NROUND * _NTILE, _VPR * _DE),
                                 jnp.float32),
            jax.ShapeDtypeStruct((_NROUND * _NTILE, _VPR, _DE),
                                 jnp.float32),
            jax.ShapeDtypeStruct((_NROUND * _NTILE, _CNTP),
                                 jnp.float32),
            jax.ShapeDtypeStruct((_NTILE, _SBLK * _S),
                                 jnp.int32),
            jax.ShapeDtypeStruct((_NTILE, _SBLK * _S),
                                 jnp.int32),
        ],
        mesh=mesh,
        scratch_types=[
            pltpu.VMEM((2, _W), jnp.int32),
            pltpu.VMEM((_S,), jnp.int32),
            pltpu.VMEM((_S,), jnp.int32),
            pltpu.VMEM((_S,), jnp.int32),
            pltpu.VMEM((_S,), jnp.int32),
            pltpu.VMEM((_S, _DE), jnp.float32),
            pltpu.VMEM((_VPR * _DE,), jnp.float32),
            pltpu.VMEM((_VPR * _DE,), jnp.float32),
            pltpu.VMEM((_CNTP,), jnp.float32),
            pltpu.VMEM_SHARED((_SCRV + _NSUB, _DE), jnp.float32),
            pltpu.SemaphoreType.DMA,
            pltpu.SemaphoreType.DMA,
            pltpu.SemaphoreType.DMA,
        ],
    )
    def agg(src_hbm, ea_hbm, min_o, max_o, sum_o, cnt_o, spl_v, spl_id,
            idx_w, sel_v, sel_id, sel_v2, sel_id2, rows, min_t, max_t,
            cnt_t, ssum, sem, wsem0, wsem1):
        core = lax.axis_index("c")
        sub = lax.axis_index("s")
        wid = core * _NSUB + sub
        tloc = sub * _VPR
        dummy = _SCRV + sub
        iota = lax.iota(jnp.int32, _L)
        ones = jnp.full((_L,), 1.0, jnp.float32)
        lane0 = iota == 0
        big = jnp.full((_L,), _BIG, jnp.float32)
        zero = jnp.zeros((_L,), jnp.float32)
        dums = jnp.full((_L,), dummy, jnp.int32)

        for r in range(_NROUND):
            sc_base = r * _RV + core * _SCRV
            lo = sc_base + tloc
            hi = jnp.minimum(lo + _VPR, (r + 1) * _RV)
            ridx = r * _NTILE + wid

            @pl.loop(0, _VPR * _DE // _L)
            def _(i):
                min_t[pl.ds(i * _L, _L)] = big
                max_t[pl.ds(i * _L, _L)] = -big

            @pl.loop(0, _CNTP // _L)
            def _(i):
                cnt_t[pl.ds(i * _L, _L)] = zero

            @pl.loop(0, _S)
            def _(i):
                rows[i, :] = zero

            @pl.loop(0, _S // _L)
            def _(i):
                sel_v[pl.ds(i * _L, _L)] = dums
                sel_id[pl.ds(i * _L, _L)] = jnp.zeros((_L,), jnp.int32)
                sel_v2[pl.ds(i * _L, _L)] = dums
                sel_id2[pl.ds(i * _L, _L)] = jnp.zeros((_L,), jnp.int32)

            nfull = _VPR // _S
            for k in range(nfull):
                pltpu.sync_copy(rows, ssum.at[pl.ds(tloc + k * _S, _S)])
            rem = _VPR - nfull * _S
            pltpu.sync_copy(rows.at[pl.ds(0, rem)],
                            ssum.at[pl.ds(tloc + nfull * _S, rem)])
            pltpu.sync_copy(rows.at[pl.ds(0, 1)], ssum.at[pl.ds(dummy, 1)])

            def flush(n_sel):
                pltpu.async_copy(ea_hbm.at[sel_id], rows, sem).wait()
                pltpu.sync_copy(rows, ssum.at[sel_v], add=True)

                ngrp = (n_sel + _L - 1) // _L

                def grp_body(gi, carry):
                    base = gi * _L
                    vv = sel_v[pl.ds(base, _L)]
                    for k in range(_L):
                        vloc = vv[k] - tloc
                        mv = jnp.full((_L,), vv[k] != dummy)
                        vsp = jnp.full((_L,), vloc, jnp.int32)
                        vidx = jnp.full((_L,), vloc * _DE, jnp.int32) + iota
                        row = rows[base + k, :]
                        m0 = plsc.load_gather(min_t, [vidx], mask=mv)
                        plsc.store_scatter(min_t, [vidx],
                                           jnp.minimum(m0, row), mask=mv)
                        m1 = plsc.load_gather(max_t, [vidx], mask=mv)
                        plsc.store_scatter(max_t, [vidx],
                                           jnp.maximum(m1, row), mask=mv)
                        plsc.addupdate_scatter(cnt_t, [vsp], ones,
                                               mask=lane0 & mv)
                    return carry

                lax.fori_loop(0, ngrp, grp_body, 0)

                @pl.loop(0, _S // _L)
                def _(i):
                    sel_v[pl.ds(i * _L, _L)] = dums

            if r == 0:
                scb = _RV + core * _SCRV
                lob = scb + tloc
                hib = jnp.minimum(lob + _VPR, _NROUND * _RV)
                spana = plsc.bitcast(
                    jnp.full((_L,), hi - lo, jnp.int32), jnp.uint32)
                spanb = plsc.bitcast(
                    jnp.full((_L,), hib - lob, jnp.int32), jnp.uint32)

                def flush2(nb):
                    pltpu.sync_copy(sel_v2,
                                    spl_v.at[wid, pl.ds(nb * _S, _S)])
                    pltpu.sync_copy(sel_id2,
                                    spl_id.at[wid, pl.ds(nb * _S, _S)])

                    @pl.loop(0, _S // _L)
                    def _(i):
                        sel_v2[pl.ds(i * _L, _L)] = dums

                def scan_buf(buf, w, st):
                    def grp_body(j, st):
                        off, off2, nb = st
                        fulla = off >= _S - _L * _U
                        fullb = off2 >= _S - _L * _U

                        @pl.when(fulla)
                        def _():
                            flush(off)

                        @pl.when(fullb)
                        def _():
                            flush2(nb)

                        off = jnp.where(fulla, 0, off)
                        off2 = jnp.where(fullb, 0, off2)
                        nb = nb + jnp.where(fullb, 1, 0)
                        base_id = w * _W + j * (_U * _L)
                        ivs, ma, mb, na, nbn = [], [], [], [], []
                        for u in range(_U):
                            iv = buf[pl.ds((j * _U + u) * _L, _L)]
                            m0 = plsc.bitcast(iv - lo,
                                              jnp.uint32) < spana
                            m1 = plsc.bitcast(iv - lob,
                                              jnp.uint32) < spanb
                            ivs.append(iv)
                            ma.append(m0)
                            mb.append(m1)
                            na.append(
                                plsc.all_reduce_population_count(m0)[0])
                            nbn.append(
                                plsc.all_reduce_population_count(m1)[0])
                        for u in range(_U):
                            ids = (base_id + u * _L) + iota
                            plsc.store_compressed(
                                sel_v.at[pl.ds(off, _L)],
                                ivs[u] - sc_base, mask=ma[u])
                            plsc.store_compressed(
                                sel_id.at[pl.ds(off, _L)], ids, mask=ma[u])
                            off = off + na[u]
                            plsc.store_compressed(
                                sel_v2.at[pl.ds(off2, _L)],
                                ivs[u] - scb, mask=mb[u])
                            plsc.store_compressed(
                                sel_id2.at[pl.ds(off2, _L)], ids,
                                mask=mb[u])
                            off2 = off2 + nbn[u]
                        return (off, off2, nb)

                    return lax.fori_loop(0, _NVREG // _U, grp_body, st)

                def start_win(w, buf, wsem):
                    pltpu.async_copy(src_hbm.at[pl.ds(w * _W, _W)], buf,
                                     wsem)

                def wait_win(w, buf, wsem):
                    pltpu.make_async_copy(src_hbm.at[pl.ds(w * _W, _W)],
                                          buf, wsem).wait()

                start_win(0, idx_w.at[0], wsem0)
                npair = _NW // 2

                def pair_body(p, st):
                    w0 = 2 * p
                    wait_win(w0, idx_w.at[0], wsem0)
                    start_win(w0 + 1, idx_w.at[1], wsem1)
                    st = scan_buf(idx_w.at[0], w0, st)
                    wait_win(w0 + 1, idx_w.at[1], wsem1)

                    @pl.when(p < npair - 1)
                    def _():
                        start_win(w0 + 2, idx_w.at[0], wsem0)

                    return scan_buf(idx_w.at[1], w0 + 1, st)

                off, off2, nb = lax.fori_loop(0, npair, pair_body,
                                              (0, 0, 0))

                @pl.when(off > 0)
                def _():
                    flush(off)

                @pl.when(off2 > 0)
                def _():
                    flush2(nb)

                nblk = nb + jnp.where(off2 > 0, 1, 0)
            else:
                def blk_body(b, carry):
                    pltpu.sync_copy(spl_v.at[wid, pl.ds(b * _S, _S)],
                                    sel_v)
                    pltpu.sync_copy(spl_id.at[wid, pl.ds(b * _S, _S)],
                                    sel_id)
                    flush(_S)
                    return carry

                lax.fori_loop(0, nblk, blk_body, 0)

            pltpu.sync_copy(min_t, min_o.at[ridx])
            pltpu.sync_copy(max_t, max_o.at[ridx])
            pltpu.sync_copy(cnt_t, cnt_o.at[ridx])
            pltpu.sync_copy(ssum.at[pl.ds(tloc, _VPR)], sum_o.at[ridx])

    return agg(src, e_attr)


_BLK = 1000


def _mlp_body(vb, mn, sm, mx, cnt, bb, g_ref, W1_ref, b1_ref, W2_ref,
              b2_ref, out_ref):
    cntv = cnt[...]
    has = cntv > 0.0
    mnv = jnp.where(has, mn[...], 0.0)
    mxv = jnp.where(has, mx[...], 0.0)
    smv = sm[...]
    meanv = smv / jnp.maximum(cntv, 1.0)

    bvals = bb[...]
    onehot = (bvals == lax.broadcasted_iota(jnp.int32, (1, _NG), 1)).astype(
        jnp.float32)
    gb = jnp.dot(onehot, g_ref[...], preferred_element_type=jnp.float32)

    W1 = W1_ref[...]
    dot = functools.partial(
        jnp.dot,
        preferred_element_type=jnp.float32,
        precision=lax.Precision.HIGHEST,
    )
    acc = dot(vb[...], W1[0:_DF])
    acc += dot(mnv, W1[_DF:_DF + _DE])
    acc += dot(meanv, W1[_DF + _DE:_DF + 2 * _DE])
    acc += dot(smv, W1[_DF + 2 * _DE:_DF + 3 * _DE])
    acc += dot(mxv, W1[_DF + 3 * _DE:_DF + 4 * _DE])
    acc += dot(gb, W1[_DF + 4 * _DE:])
    h = jnp.maximum(acc + b1_ref[...], 0.0)
    out_ref[...] = dot(h, W2_ref[...]) + b2_ref[...]


def _mlp(v_attr, mn, sm, mx, cnt, batch, g, W1, b1, W2, b2):
    nblk = _N // _BLK
    row = lambda i: (i, 0)
    rep = lambda i: (0, 0)
    return pl.pallas_call(
        _mlp_body,
        grid=(nblk,),
        in_specs=[
            pl.BlockSpec((_BLK, _DF), row),
            pl.BlockSpec((_BLK, _DE), row),
            pl.BlockSpec((_BLK, _DE), row),
            pl.BlockSpec((_BLK, _DE), row),
            pl.BlockSpec((_BLK, 1), row),
            pl.BlockSpec((_BLK, 1), row),
            pl.BlockSpec((_NG, _DG), rep),
            pl.BlockSpec((_DF + 4 * _DE + _DG, _NH), rep),
            pl.BlockSpec((1, _NH), rep),
            pl.BlockSpec((_NH, _NO), rep),
            pl.BlockSpec((1, _NO), rep),
        ],
        out_specs=pl.BlockSpec((_BLK, _NO), row),
        out_shape=jax.ShapeDtypeStruct((_N, _NO), jnp.float32),
    )(v_attr, mn, sm, mx, cnt, batch, g, W1, b1, W2, b2)


def kernel(v_attr, edgeij_pair, e_attr, g, batch, W1, b1, W2, b2):
    src = edgeij_pair[0]
    mn, mx, sm, cnt, _unused_v, _unused_id = _sc_agg(src, e_attr)
    trim = lambda x: (x.reshape(_NROUND, _NTILE * _VPR,
                                _DE)[:, :_RV].reshape(_N, _DE))
    mn = trim(mn)
    mx = trim(mx)
    sm = trim(sm)
    cnt2 = cnt[:, :_VPR].reshape(_NROUND,
                                 _NTILE * _VPR)[:, :_RV].reshape(_N, 1)
    return _mlp(v_attr, mn, sm, mx, cnt2,
                batch[:, None].astype(jnp.int32), g, W1, b1[None, :], W2,
                b2[None, :])


# speedup vs baseline: 1.0103x; 1.0103x over previous
import dataclasses
import functools

import jax
import jax.numpy as jnp
from jax import lax
from jax.experimental import pallas as pl
from jax.experimental.pallas import tpu as pltpu
from jax.experimental.pallas import tpu_sc as plsc

_N = 100000
_E = 3200000
_DF = 128
_DE = 16
_NG = 16
_DG = 32
_NH = 128
_NO = 128

_L = 16
_NSUB = 16
_NCORE = 2
_NTILE = _NSUB * _NCORE
_NROUND = 2
_RV = _N // _NROUND
_VPR = 1563
_SCRV = _NSUB * _VPR
_W = 4000
_NW = _E // _W
_NVREG = _W // _L
_S = 1024
_U = 10
_SBLK = 3750
_CNTP = 1568
_BIG = 3.0e38


def _sc_agg(src, e_attr):
    mesh = plsc.VectorSubcoreMesh(core_axis_name="c", subcore_axis_name="s")
    cp = pltpu.CompilerParams()
    if "needs_layout_passes" in pltpu.CompilerParams.__dataclass_fields__:
        cp = dataclasses.replace(cp, needs_layout_passes=False)
    cp = dataclasses.replace(cp, use_tc_tiling_on_sc=False)

    @functools.partial(
        pl.kernel,
        compiler_params=cp,
        out_type=[
            jax.ShapeDtypeStruct((_NROUND * _NTILE, _VPR * _DE),
                                 jnp.float32),
            jax.ShapeDtypeStruct((_NROUND * _NTILE, _VPR * _DE),
                                 jnp.float32),
            jax.ShapeDtypeStruct((_NROUND * _NTILE, _VPR, _DE),
                                 jnp.float32),
            jax.ShapeDtypeStruct((_NROUND * _NTILE, _CNTP),
                                 jnp.float32),
            jax.ShapeDtypeStruct((_NTILE, _SBLK * _S),
                                 jnp.int32),
            jax.ShapeDtypeStruct((_NTILE, _SBLK * _S),
                                 jnp.int32),
        ],
        mesh=mesh,
        scratch_types=[
            pltpu.VMEM((2, _W), jnp.int32),
            pltpu.VMEM((_S,), jnp.int32),
            pltpu.VMEM((_S,), jnp.int32),
            pltpu.VMEM((_S,), jnp.int32),
            pltpu.VMEM((_S,), jnp.int32),
            pltpu.VMEM((_S, _DE), jnp.float32),
            pltpu.VMEM((_VPR * _DE,), jnp.float32),
            pltpu.VMEM((_VPR * _DE,), jnp.float32),
            pltpu.VMEM((_CNTP,), jnp.float32),
            pltpu.VMEM_SHARED((_SCRV + _NSUB, _DE), jnp.float32),
            pltpu.SemaphoreType.DMA,
            pltpu.SemaphoreType.DMA,
            pltpu.SemaphoreType.DMA,
            pltpu.SemaphoreType.DMA,
        ],
    )
    def agg(src_hbm, ea_hbm, min_o, max_o, sum_o, cnt_o, spl_v, spl_id,
            idx_w, sel_v, sel_id, sel_v2, sel_id2, rows, min_t, max_t,
            cnt_t, ssum, sem, gsem1, wsem0, wsem1):
        core = lax.axis_index("c")
        sub = lax.axis_index("s")
        wid = core * _NSUB + sub
        tloc = sub * _VPR
        dummy = _SCRV + sub
        iota = lax.iota(jnp.int32, _L)
        ones = jnp.full((_L,), 1.0, jnp.float32)
        lane0 = iota == 0
        big = jnp.full((_L,), _BIG, jnp.float32)
        zero = jnp.zeros((_L,), jnp.float32)
        dums = jnp.full((_L,), dummy, jnp.int32)

        for r in range(_NROUND):
            sc_base = r * _RV + core * _SCRV
            lo = sc_base + tloc
            hi = jnp.minimum(lo + _VPR, (r + 1) * _RV)
            ridx = r * _NTILE + wid

            @pl.loop(0, _VPR * _DE // _L)
            def _(i):
                min_t[pl.ds(i * _L, _L)] = big
                max_t[pl.ds(i * _L, _L)] = -big

            @pl.loop(0, _CNTP // _L)
            def _(i):
                cnt_t[pl.ds(i * _L, _L)] = zero

            @pl.loop(0, _S)
            def _(i):
                rows[i, :] = zero

            @pl.loop(0, _S // _L)
            def _(i):
                sel_v[pl.ds(i * _L, _L)] = dums
                sel_id[pl.ds(i * _L, _L)] = jnp.zeros((_L,), jnp.int32)
                sel_v2[pl.ds(i * _L, _L)] = dums
                sel_id2[pl.ds(i * _L, _L)] = jnp.zeros((_L,), jnp.int32)

            nfull = _VPR // _S
            for k in range(nfull):
                pltpu.sync_copy(rows, ssum.at[pl.ds(tloc + k * _S, _S)])
            rem = _VPR - nfull * _S
            pltpu.sync_copy(rows.at[pl.ds(0, rem)],
                            ssum.at[pl.ds(tloc + nfull * _S, rem)])
            pltpu.sync_copy(rows.at[pl.ds(0, 1)], ssum.at[pl.ds(dummy, 1)])

            def flush(n_sel):
                h = _S // 2
                cp0 = pltpu.async_copy(ea_hbm.at[sel_id.at[pl.ds(0, h)]],
                                       rows.at[pl.ds(0, h)], sem)
                cp1 = pltpu.async_copy(ea_hbm.at[sel_id.at[pl.ds(h, h)]],
                                       rows.at[pl.ds(h, h)], gsem1)

                def grp_body(gi, carry):
                    base = gi * _L
                    vv = sel_v[pl.ds(base, _L)]
                    for k in range(_L):
                        vloc = vv[k] - tloc
                        mv = jnp.full((_L,), vv[k] != dummy)
                        vsp = jnp.full((_L,), vloc, jnp.int32)
                        vidx = jnp.full((_L,), vloc * _DE, jnp.int32) + iota
                        row = rows[base + k, :]
                        m0 = plsc.load_gather(min_t, [vidx], mask=mv)
                        plsc.store_scatter(min_t, [vidx],
                                           jnp.minimum(m0, row), mask=mv)
                        m1 = plsc.load_gather(max_t, [vidx], mask=mv)
                        plsc.store_scatter(max_t, [vidx],
                                           jnp.maximum(m1, row), mask=mv)
                        plsc.addupdate_scatter(cnt_t, [vsp], ones,
                                               mask=lane0 & mv)
                    return carry

                hg = h // _L
                ngrp0 = (jnp.minimum(n_sel, h) + _L - 1) // _L
                ngrp1 = (jnp.maximum(n_sel - h, 0) + _L - 1) // _L
                cp0.wait()
                pltpu.sync_copy(rows.at[pl.ds(0, h)],
                                ssum.at[sel_v.at[pl.ds(0, h)]], add=True)
                lax.fori_loop(0, ngrp0, grp_body, 0)
                cp1.wait()
                pltpu.sync_copy(rows.at[pl.ds(h, h)],
                                ssum.at[sel_v.at[pl.ds(h, h)]], add=True)
                lax.fori_loop(hg, hg + ngrp1, grp_body, 0)

                @pl.loop(0, _S // _L)
                def _(i):
                    sel_v[pl.ds(i * _L, _L)] = dums

            if r == 0:
                scb = _RV + core * _SCRV
                lob = scb + tloc
                hib = jnp.minimum(lob + _VPR, _NROUND * _RV)
                spana = plsc.bitcast(
                    jnp.full((_L,), hi - lo, jnp.int32), jnp.uint32)
                spanb = plsc.bitcast(
                    jnp.full((_L,), hib - lob, jnp.int32), jnp.uint32)

                def flush2(nb):
                    pltpu.sync_copy(sel_v2,
                                    spl_v.at[wid, pl.ds(nb * _S, _S)])
                    pltpu.sync_copy(sel_id2,
                                    spl_id.at[wid, pl.ds(nb * _S, _S)])

                    @pl.loop(0, _S // _L)
                    def _(i):
                        sel_v2[pl.ds(i * _L, _L)] = dums

                def scan_buf(buf, w, st):
                    def grp_body(j, st):
                        off, off2, nb = st
                        fulla = off >= _S - _L * _U
                        fullb = off2 >= _S - _L * _U

                        @pl.when(fulla)
                        def _():
                            flush(off)

                        @pl.when(fullb)
                        def _():
                            flush2(nb)

                        off = jnp.where(fulla, 0, off)
                        off2 = jnp.where(fullb, 0, off2)
                        nb = nb + jnp.where(fullb, 1, 0)
                        base_id = w * _W + j * (_U * _L)
                        ivs, ma, mb, na, nbn = [], [], [], [], []
                        for u in range(_U):
                            iv = buf[pl.ds((j * _U + u) * _L, _L)]
                            m0 = plsc.bitcast(iv - lo,
                                              jnp.uint32) < spana
                            m1 = plsc.bitcast(iv - lob,
                                              jnp.uint32) < spanb
                            ivs.append(iv)
                            ma.append(m0)
                            mb.append(m1)
                            na.append(
                                plsc.all_reduce_population_count(m0)[0])
                            nbn.append(
                                plsc.all_reduce_population_count(m1)[0])
                        for u in range(_U):
                            ids = (base_id + u * _L) + iota
                            plsc.store_compressed(
                                sel_v.at[pl.ds(off, _L)],
                                ivs[u] - sc_base, mask=ma[u])
                            plsc.store_compressed(
                                sel_id.at[pl.ds(off, _L)], ids, mask=ma[u])
                            off = off + na[u]
                            plsc.store_compressed(
                                sel_v2.at[pl.ds(off2, _L)],
                                ivs[u] - scb, mask=mb[u])
                            plsc.store_compressed(
                                sel_id2.at[pl.ds(off2, _L)], ids,
                                mask=mb[u])
                            off2 = off2 + nbn[u]
                        return (off, off2, nb)

                    return lax.fori_loop(0, _NVREG // _U, grp_body, st)

                def start_win(w, buf, wsem):
                    pltpu.async_copy(src_hbm.at[pl.ds(w * _W, _W)], buf,
                                     wsem)

                def wait_win(w, buf, wsem):
                    pltpu.make_async_copy(src_hbm.at[pl.ds(w * _W, _W)],
                                          buf, wsem).wait()

                start_win(0, idx_w.at[0], wsem0)
                npair = _NW // 2

                def pair_body(p, st):
                    w0 = 2 * p
                    wait_win(w0, idx_w.at[0], wsem0)
                    start_win(w0 + 1, idx_w.at[1], wsem1)
                    st = scan_buf(idx_w.at[0], w0, st)
                    wait_win(w0 + 1, idx_w.at[1], wsem1)

                    @pl.when(p < npair - 1)
                    def _():
                        start_win(w0 + 2, idx_w.at[0], wsem0)

                    return scan_buf(idx_w.at[1], w0 + 1, st)

                off, off2, nb = lax.fori_loop(0, npair, pair_body,
                                              (0, 0, 0))

                @pl.when(off > 0)
                def _():
                    flush(off)

                @pl.when(off2 > 0)
                def _():
                    flush2(nb)

                nblk = nb + jnp.where(off2 > 0, 1, 0)
            else:
                def blk_body(b, carry):
                    pltpu.sync_copy(spl_v.at[wid, pl.ds(b * _S, _S)],
                                    sel_v)
                    pltpu.sync_copy(spl_id.at[wid, pl.ds(b * _S, _S)],
                                    sel_id)
                    flush(_S)
                    return carry

                lax.fori_loop(0, nblk, blk_body, 0)

            pltpu.sync_copy(min_t, min_o.at[ridx])
            pltpu.sync_copy(max_t, max_o.at[ridx])
            pltpu.sync_copy(cnt_t, cnt_o.at[ridx])
            pltpu.sync_copy(ssum.at[pl.ds(tloc, _VPR)], sum_o.at[ridx])

    return agg(src, e_attr)


_BLK = 1000


def _mlp_body(vb, mn, sm, mx, cnt, bb, g_ref, W1_ref, b1_ref, W2_ref,
              b2_ref, out_ref):
    cntv = cnt[...]
    has = cntv > 0.0
    mnv = jnp.where(has, mn[...], 0.0)
    mxv = jnp.where(has, mx[...], 0.0)
    smv = sm[...]
    meanv = smv / jnp.maximum(cntv, 1.0)

    bvals = bb[...]
    onehot = (bvals == lax.broadcasted_iota(jnp.int32, (1, _NG), 1)).astype(
        jnp.float32)
    gb = jnp.dot(onehot, g_ref[...], preferred_element_type=jnp.float32)

    W1 = W1_ref[...]
    dot = functools.partial(
        jnp.dot,
        preferred_element_type=jnp.float32,
        precision=lax.Precision.HIGHEST,
    )
    acc = dot(vb[...], W1[0:_DF])
    acc += dot(mnv, W1[_DF:_DF + _DE])
    acc += dot(meanv, W1[_DF + _DE:_DF + 2 * _DE])
    acc += dot(smv, W1[_DF + 2 * _DE:_DF + 3 * _DE])
    acc += dot(mxv, W1[_DF + 3 * _DE:_DF + 4 * _DE])
    acc += dot(gb, W1[_DF + 4 * _DE:])
    h = jnp.maximum(acc + b1_ref[...], 0.0)
    out_ref[...] = dot(h, W2_ref[...]) + b2_ref[...]


def _mlp(v_attr, mn, sm, mx, cnt, batch, g, W1, b1, W2, b2):
    nblk = _N // _BLK
    row = lambda i: (i, 0)
    rep = lambda i: (0, 0)
    return pl.pallas_call(
        _mlp_body,
        grid=(nblk,),
        in_specs=[
            pl.BlockSpec((_BLK, _DF), row),
            pl.BlockSpec((_BLK, _DE), row),
            pl.BlockSpec((_BLK, _DE), row),
            pl.BlockSpec((_BLK, _DE), row),
            pl.BlockSpec((_BLK, 1), row),
            pl.BlockSpec((_BLK, 1), row),
            pl.BlockSpec((_NG, _DG), rep),
            pl.BlockSpec((_DF + 4 * _DE + _DG, _NH), rep),
            pl.BlockSpec((1, _NH), rep),
            pl.BlockSpec((_NH, _NO), rep),
            pl.BlockSpec((1, _NO), rep),
        ],
        out_specs=pl.BlockSpec((_BLK, _NO), row),
        out_shape=jax.ShapeDtypeStruct((_N, _NO), jnp.float32),
    )(v_attr, mn, sm, mx, cnt, batch, g, W1, b1, W2, b2)


def kernel(v_attr, edgeij_pair, e_attr, g, batch, W1, b1, W2, b2):
    src = edgeij_pair[0]
    mn, mx, sm, cnt, _unused_v, _unused_id = _sc_agg(src, e_attr)
    trim = lambda x: (x.reshape(_NROUND, _NTILE * _VPR,
                                _DE)[:, :_RV].reshape(_N, _DE))
    mn = trim(mn)
    mx = trim(mx)
    sm = trim(sm)
    cnt2 = cnt[:, :_VPR].reshape(_NROUND,
                                 _NTILE * _VPR)[:, :_RV].reshape(_N, 1)
    return _mlp(v_attr, mn, sm, mx, cnt2,
                batch[:, None].astype(jnp.int32), g, W1, b1[None, :], W2,
                b2[None, :])


# speedup vs baseline: 1.1936x; 1.1815x over previous
import dataclasses
import functools

import jax
import jax.numpy as jnp
from jax import lax
from jax.experimental import pallas as pl
from jax.experimental.pallas import tpu as pltpu
from jax.experimental.pallas import tpu_sc as plsc

_N = 100000
_E = 3200000
_DF = 128
_DE = 16
_NG = 16
_DG = 32
_NH = 128
_NO = 128

_L = 16
_NSUB = 16
_NCORE = 2
_NTILE = _NSUB * _NCORE
_NROUND = 2
_RV = _N // _NROUND
_VPR = 1563
_SCRV = _NSUB * _VPR
_W = 4000
_NW = _E // _W
_NVREG = _W // _L
_S = 2048
_U = 10
_SBLK = 1700
_CNTP = 1568
_BIG = 3.0e38


def _sc_agg(src, e_attr):
    mesh = plsc.VectorSubcoreMesh(core_axis_name="c", subcore_axis_name="s")
    cp = pltpu.CompilerParams()
    if "needs_layout_passes" in pltpu.CompilerParams.__dataclass_fields__:
        cp = dataclasses.replace(cp, needs_layout_passes=False)
    cp = dataclasses.replace(cp, use_tc_tiling_on_sc=False)

    @functools.partial(
        pl.kernel,
        compiler_params=cp,
        out_type=[
            jax.ShapeDtypeStruct((_NROUND * _NTILE, _VPR * _DE),
                                 jnp.float32),
            jax.ShapeDtypeStruct((_NROUND * _NTILE, _VPR * _DE),
                                 jnp.float32),
            jax.ShapeDtypeStruct((_NROUND * _NTILE, _VPR, _DE),
                                 jnp.float32),
            jax.ShapeDtypeStruct((_NROUND * _NTILE, _CNTP),
                                 jnp.float32),
            jax.ShapeDtypeStruct((_NTILE, _SBLK * _S),
                                 jnp.int32),
            jax.ShapeDtypeStruct((_NTILE, _SBLK * _S),
                                 jnp.int32),
        ],
        mesh=mesh,
        scratch_types=[
            pltpu.VMEM((2, _W), jnp.int32),
            pltpu.VMEM((_S,), jnp.int32),
            pltpu.VMEM((_S,), jnp.int32),
            pltpu.VMEM((_S,), jnp.int32),
            pltpu.VMEM((_S,), jnp.int32),
            pltpu.VMEM((_S, _DE), jnp.float32),
            pltpu.VMEM((_VPR * _DE,), jnp.float32),
            pltpu.VMEM((_VPR * _DE,), jnp.float32),
            pltpu.VMEM((_CNTP,), jnp.float32),
            pltpu.VMEM_SHARED((_SCRV + _NSUB, _DE), jnp.float32),
            pltpu.SemaphoreType.DMA,
            pltpu.SemaphoreType.DMA,
            pltpu.SemaphoreType.DMA,
            pltpu.SemaphoreType.DMA,
        ],
    )
    def agg(src_hbm, ea_hbm, min_o, max_o, sum_o, cnt_o, spl_v, spl_id,
            idx_w, sel_v, sel_id, sel_v2, sel_id2, rows, min_t, max_t,
            cnt_t, ssum, sem, gsem1, wsem0, wsem1):
        core = lax.axis_index("c")
        sub = lax.axis_index("s")
        wid = core * _NSUB + sub
        tloc = sub * _VPR
        dummy = _SCRV + sub
        iota = lax.iota(jnp.int32, _L)
        ones = jnp.full((_L,), 1.0, jnp.float32)
        lane0 = iota == 0
        big = jnp.full((_L,), _BIG, jnp.float32)
        zero = jnp.zeros((_L,), jnp.float32)
        dums = jnp.full((_L,), dummy, jnp.int32)

        for r in range(_NROUND):
            sc_base = r * _RV + core * _SCRV
            lo = sc_base + tloc
            hi = jnp.minimum(lo + _VPR, (r + 1) * _RV)
            ridx = r * _NTILE + wid

            @pl.loop(0, _VPR * _DE // _L)
            def _(i):
                min_t[pl.ds(i * _L, _L)] = big
                max_t[pl.ds(i * _L, _L)] = -big

            @pl.loop(0, _CNTP // _L)
            def _(i):
                cnt_t[pl.ds(i * _L, _L)] = zero

            @pl.loop(0, _S)
            def _(i):
                rows[i, :] = zero

            @pl.loop(0, _S // _L)
            def _(i):
                sel_v[pl.ds(i * _L, _L)] = dums
                sel_id[pl.ds(i * _L, _L)] = jnp.zeros((_L,), jnp.int32)
                sel_v2[pl.ds(i * _L, _L)] = dums
                sel_id2[pl.ds(i * _L, _L)] = jnp.zeros((_L,), jnp.int32)

            nfull = _VPR // _S
            for k in range(nfull):
                pltpu.sync_copy(rows, ssum.at[pl.ds(tloc + k * _S, _S)])
            rem = _VPR - nfull * _S
            pltpu.sync_copy(rows.at[pl.ds(0, rem)],
                            ssum.at[pl.ds(tloc + nfull * _S, rem)])
            pltpu.sync_copy(rows.at[pl.ds(0, 1)], ssum.at[pl.ds(dummy, 1)])

            def flush(n_sel):
                h = _S // 2
                cp0 = pltpu.async_copy(ea_hbm.at[sel_id.at[pl.ds(0, h)]],
                                       rows.at[pl.ds(0, h)], sem)
                cp1 = pltpu.async_copy(ea_hbm.at[sel_id.at[pl.ds(h, h)]],
                                       rows.at[pl.ds(h, h)], gsem1)

                def grp_body(gi, carry):
                    base = gi * _L
                    vv = sel_v[pl.ds(base, _L)]
                    for k in range(_L):
                        vloc = vv[k] - tloc
                        mv = jnp.full((_L,), vv[k] != dummy)
                        vsp = jnp.full((_L,), vloc, jnp.int32)
                        vidx = jnp.full((_L,), vloc * _DE, jnp.int32) + iota
                        row = rows[base + k, :]
                        m0 = plsc.load_gather(min_t, [vidx], mask=mv)
                        plsc.store_scatter(min_t, [vidx],
                                           jnp.minimum(m0, row), mask=mv)
                        m1 = plsc.load_gather(max_t, [vidx], mask=mv)
                        plsc.store_scatter(max_t, [vidx],
                                           jnp.maximum(m1, row), mask=mv)
                        plsc.addupdate_scatter(cnt_t, [vsp], ones,
                                               mask=lane0 & mv)
                    return carry

                hg = h // _L
                ngrp0 = (jnp.minimum(n_sel, h) + _L - 1) // _L
                ngrp1 = (jnp.maximum(n_sel - h, 0) + _L - 1) // _L
                cp0.wait()
                pltpu.sync_copy(rows.at[pl.ds(0, h)],
                                ssum.at[sel_v.at[pl.ds(0, h)]], add=True)
                lax.fori_loop(0, ngrp0, grp_body, 0)
                cp1.wait()
                pltpu.sync_copy(rows.at[pl.ds(h, h)],
                                ssum.at[sel_v.at[pl.ds(h, h)]], add=True)
                lax.fori_loop(hg, hg + ngrp1, grp_body, 0)

                @pl.loop(0, _S // _L)
                def _(i):
                    sel_v[pl.ds(i * _L, _L)] = dums

            if r == 0:
                scb = _RV + core * _SCRV
                lob = scb + tloc
                hib = jnp.minimum(lob + _VPR, _NROUND * _RV)
                spana = plsc.bitcast(
                    jnp.full((_L,), hi - lo, jnp.int32), jnp.uint32)
                spanb = plsc.bitcast(
                    jnp.full((_L,), hib - lob, jnp.int32), jnp.uint32)

                def flush2(nb):
                    pltpu.sync_copy(sel_v2,
                                    spl_v.at[wid, pl.ds(nb * _S, _S)])
                    pltpu.sync_copy(sel_id2,
                                    spl_id.at[wid, pl.ds(nb * _S, _S)])

                    @pl.loop(0, _S // _L)
                    def _(i):
                        sel_v2[pl.ds(i * _L, _L)] = dums

                def scan_buf(buf, w, st):
                    def grp_body(j, st):
                        off, off2, nb = st
                        fulla = off >= _S - _L * _U
                        fullb = off2 >= _S - _L * _U

                        @pl.when(fulla)
                        def _():
                            flush(off)

                        @pl.when(fullb)
                        def _():
                            flush2(nb)

                        off = jnp.where(fulla, 0, off)
                        off2 = jnp.where(fullb, 0, off2)
                        nb = nb + jnp.where(fullb, 1, 0)
                        base_id = w * _W + j * (_U * _L)
                        ivs, ma, mb, na, nbn = [], [], [], [], []
                        for u in range(_U):
                            iv = buf[pl.ds((j * _U + u) * _L, _L)]
                            m0 = plsc.bitcast(iv - lo,
                                              jnp.uint32) < spana
                            m1 = plsc.bitcast(iv - lob,
                                              jnp.uint32) < spanb
                            ivs.append(iv)
                            ma.append(m0)
                            mb.append(m1)
                            na.append(
                                plsc.all_reduce_population_count(m0)[0])
                            nbn.append(
                                plsc.all_reduce_population_count(m1)[0])
                        for u in range(_U):
                            ids = (base_id + u * _L) + iota
                            plsc.store_compressed(
                                sel_v.at[pl.ds(off, _L)],
                                ivs[u] - sc_base, mask=ma[u])
                            plsc.store_compressed(
                                sel_id.at[pl.ds(off, _L)], ids, mask=ma[u])
                            off = off + na[u]
                            plsc.store_compressed(
                                sel_v2.at[pl.ds(off2, _L)],
                                ivs[u] - scb, mask=mb[u])
                            plsc.store_compressed(
                                sel_id2.at[pl.ds(off2, _L)], ids,
                                mask=mb[u])
                            off2 = off2 + nbn[u]
                        return (off, off2, nb)

                    return lax.fori_loop(0, _NVREG // _U, grp_body, st)

                def start_win(w, buf, wsem):
                    pltpu.async_copy(src_hbm.at[pl.ds(w * _W, _W)], buf,
                                     wsem)

                def wait_win(w, buf, wsem):
                    pltpu.make_async_copy(src_hbm.at[pl.ds(w * _W, _W)],
                                          buf, wsem).wait()

                start_win(0, idx_w.at[0], wsem0)
                npair = _NW // 2

                def pair_body(p, st):
                    w0 = 2 * p
                    wait_win(w0, idx_w.at[0], wsem0)
                    start_win(w0 + 1, idx_w.at[1], wsem1)
                    st = scan_buf(idx_w.at[0], w0, st)
                    wait_win(w0 + 1, idx_w.at[1], wsem1)

                    @pl.when(p < npair - 1)
                    def _():
                        start_win(w0 + 2, idx_w.at[0], wsem0)

                    return scan_buf(idx_w.at[1], w0 + 1, st)

                off, off2, nb = lax.fori_loop(0, npair, pair_body,
                                              (0, 0, 0))

                @pl.when(off > 0)
                def _():
                    flush(off)

                @pl.when(off2 > 0)
                def _():
                    flush2(nb)

                nblk = nb + jnp.where(off2 > 0, 1, 0)
            else:
                def blk_body(b, carry):
                    pltpu.sync_copy(spl_v.at[wid, pl.ds(b * _S, _S)],
                                    sel_v)
                    pltpu.sync_copy(spl_id.at[wid, pl.ds(b * _S, _S)],
                                    sel_id)
                    flush(_S)
                    return carry

                lax.fori_loop(0, nblk, blk_body, 0)

            pltpu.sync_copy(min_t, min_o.at[ridx])
            pltpu.sync_copy(max_t, max_o.at[ridx])
            pltpu.sync_copy(cnt_t, cnt_o.at[ridx])
            pltpu.sync_copy(ssum.at[pl.ds(tloc, _VPR)], sum_o.at[ridx])

    return agg(src, e_attr)


_BLK = 1000


def _mlp_body(vb, mn, sm, mx, cnt, bb, g_ref, W1_ref, b1_ref, W2_ref,
              b2_ref, out_ref):
    cntv = cnt[...]
    has = cntv > 0.0
    mnv = jnp.where(has, mn[...], 0.0)
    mxv = jnp.where(has, mx[...], 0.0)
    smv = sm[...]
    meanv = smv / jnp.maximum(cntv, 1.0)

    bvals = bb[...]
    onehot = (bvals == lax.broadcasted_iota(jnp.int32, (1, _NG), 1)).astype(
        jnp.float32)
    gb = jnp.dot(onehot, g_ref[...], preferred_element_type=jnp.float32)

    W1 = W1_ref[...]
    dot = functools.partial(
        jnp.dot,
        preferred_element_type=jnp.float32,
        precision=lax.Precision.HIGHEST,
    )
    acc = dot(vb[...], W1[0:_DF])
    acc += dot(mnv, W1[_DF:_DF + _DE])
    acc += dot(meanv, W1[_DF + _DE:_DF + 2 * _DE])
    acc += dot(smv, W1[_DF + 2 * _DE:_DF + 3 * _DE])
    acc += dot(mxv, W1[_DF + 3 * _DE:_DF + 4 * _DE])
    acc += dot(gb, W1[_DF + 4 * _DE:])
    h = jnp.maximum(acc + b1_ref[...], 0.0)
    out_ref[...] = dot(h, W2_ref[...]) + b2_ref[...]


def _mlp(v_attr, mn, sm, mx, cnt, batch, g, W1, b1, W2, b2):
    nblk = _N // _BLK
    row = lambda i: (i, 0)
    rep = lambda i: (0, 0)
    return pl.pallas_call(
        _mlp_body,
        grid=(nblk,),
        in_specs=[
            pl.BlockSpec((_BLK, _DF), row),
            pl.BlockSpec((_BLK, _DE), row),
            pl.BlockSpec((_BLK, _DE), row),
            pl.BlockSpec((_BLK, _DE), row),
            pl.BlockSpec((_BLK, 1), row),
            pl.BlockSpec((_BLK, 1), row),
            pl.BlockSpec((_NG, _DG), rep),
            pl.BlockSpec((_DF + 4 * _DE + _DG, _NH), rep),
            pl.BlockSpec((1, _NH), rep),
            pl.BlockSpec((_NH, _NO), rep),
            pl.BlockSpec((1, _NO), rep),
        ],
        out_specs=pl.BlockSpec((_BLK, _NO), row),
        out_shape=jax.ShapeDtypeStruct((_N, _NO), jnp.float32),
    )(v_attr, mn, sm, mx, cnt, batch, g, W1, b1, W2, b2)


def kernel(v_attr, edgeij_pair, e_attr, g, batch, W1, b1, W2, b2):
    src = edgeij_pair[0]
    mn, mx, sm, cnt, _unused_v, _unused_id = _sc_agg(src, e_attr)
    trim = lambda x: (x.reshape(_NROUND, _NTILE * _VPR,
                                _DE)[:, :_RV].reshape(_N, _DE))
    mn = trim(mn)
    mx = trim(mx)
    sm = trim(sm)
    cnt2 = cnt[:, :_VPR].reshape(_NROUND,
                                 _NTILE * _VPR)[:, :_RV].reshape(_N, 1)
    return _mlp(v_attr, mn, sm, mx, cnt2,
                batch[:, None].astype(jnp.int32), g, W1, b1[None, :], W2,
                b2[None, :])
